# baseline (speedup 1.0000x reference)
import functools

import jax
import jax.numpy as jnp
from jax import lax
from jax.experimental import pallas as pl
from jax.experimental.pallas import tpu as pltpu
from jax.experimental.pallas import tpu_sc as plsc

G = 1024
COMMIT = 0.25

_NC = 2
_NS = 16
_NT = _NC * _NS
_CHUNK = 128
_NCHUNK = 25
_NPAD = _NT * _NCHUNK * _CHUNK
_PAD = G


def _vq_body(x_ref, sc_ref, cb_ref, cr_ref, cmt_ref):
    i = pl.program_id(0)

    @pl.when(i == 0)
    def _init():
        cmt_ref[...] = jnp.zeros_like(cmt_ref)

    x = x_ref[...]
    cb = cb_ref[...]
    K = cb.shape[0]

    x_sq = jnp.sum(x * x, axis=1, keepdims=True)
    cb_sq = jnp.sum(cb * cb, axis=1)[None, :]
    xc = lax.dot_general(x, cb, (((1,), (1,)), ((), ())),
                         preferred_element_type=jnp.float32)
    dist = x_sq - 2.0 * xc + cb_sq

    m = jnp.min(dist, axis=1, keepdims=True)
    cmt_ref[...] += jnp.sum(m).reshape(1, 1)

    onehot_k = (dist == m).astype(jnp.float32)
    nsel = jnp.sum(onehot_k, axis=1, keepdims=True)
    q = lax.dot_general(onehot_k, cb, (((1,), (0,)), ((), ())),
                        preferred_element_type=jnp.float32) / nsel

    score = sc_ref[...]
    r = x + q
    c = r * score
    cr_ref[...] = jnp.concatenate([c, r], axis=1)


def _cnt_sc_body(idx_hbm, one_hbm, zer_hbm, cnt_hbm, onesb, idxb, cnt_sh):
    cid = lax.axis_index("c")
    sid = lax.axis_index("s")
    wid = cid * _NS + sid

    r0 = sid * 64
    pltpu.sync_copy(zer_hbm, cnt_sh.at[pl.ds(r0, 64)])

    @pl.when(sid == _NS - 1)
    def _pad_row():
        pltpu.sync_copy(zer_hbm.at[pl.ds(0, 1)], cnt_sh.at[pl.ds(G, 1)])

    pltpu.sync_copy(idx_hbm.at[wid], idxb)
    pltpu.sync_copy(one_hbm, onesb)

    plsc.subcore_barrier()

    for t in range(_NCHUNK):
        pltpu.sync_copy(onesb, cnt_sh.at[idxb.at[t]], add=True)

    plsc.subcore_barrier()

    pltpu.sync_copy(cnt_sh.at[pl.ds(r0, 64)], cnt_hbm.at[cid, pl.ds(r0, 64)])


def _pool_sc_body(cr_hbm, idx_hbm, zer_hbm, acc_hbm, buf, idxb, acc_sh):
    cid = lax.axis_index("c")
    sid = lax.axis_index("s")
    wid = cid * _NS + sid

    r0 = sid * 64
    pltpu.sync_copy(zer_hbm, acc_sh.at[pl.ds(r0, 64)])

    @pl.when(sid == _NS - 1)
    def _pad_row():
        pltpu.sync_copy(zer_hbm.at[pl.ds(0, 1)], acc_sh.at[pl.ds(G, 1)])

    pltpu.sync_copy(idx_hbm.at[wid], idxb)

    plsc.subcore_barrier()

    base = wid * (_NCHUNK * _CHUNK)
    for t in range(_NCHUNK):
        row0 = base + t * _CHUNK
        pltpu.sync_copy(cr_hbm.at[pl.ds(row0, _CHUNK)], buf)
        pltpu.sync_copy(buf, acc_sh.at[idxb.at[t]], add=True)

    plsc.subcore_barrier()

    pltpu.sync_copy(acc_sh.at[pl.ds(r0, 64)], acc_hbm.at[cid, pl.ds(r0, 64)])


def _classifier_body(acc_ref, cnt_ref, cmt_ref, w1_ref, b1_ref, g_ref, be_ref,
                     w2_ref, b2_ref, logit_ref, cg_ref, sg_ref, cmt_out_ref,
                     *, n_total, d):
    acc = acc_ref[0] + acc_ref[1]
    denom = jnp.maximum(cnt_ref[0, :, :1] + cnt_ref[1, :, :1], 1.0)
    cg = acc[:, :d] / denom
    rg = acc[:, d:] / denom
    sg = rg - cg
    cg_ref[...] = cg
    sg_ref[...] = sg

    h = lax.dot_general(cg, w1_ref[...], (((1,), (0,)), ((), ())),
                        preferred_element_type=jnp.float32) + b1_ref[...]
    gn = h.shape[0]
    mu = jnp.sum(h, axis=0, keepdims=True) / gn
    var = jnp.sum((h - mu) ** 2, axis=0, keepdims=True) / gn
    hn = (h - mu) / jnp.sqrt(var + 1e-5) * g_ref[...] + be_ref[...]
    hr = jnp.maximum(hn, 0.0)
    logit_ref[...] = lax.dot_general(hr, w2_ref[...], (((1,), (0,)), ((), ())),
                                     preferred_element_type=jnp.float32) + b2_ref[...]

    cmt_out_ref[...] = cmt_ref[...] * (COMMIT / (n_total * d))


def kernel(node_feat, score, codebook, W1, b1, gamma, beta, W2, b2, batch):
    N, D = node_feat.shape
    K = codebook.shape[0]
    H = W1.shape[1]
    D2 = 2 * D

    B = 10000
    n_blocks = N // B

    cr, cmt = pl.pallas_call(
        _vq_body,
        grid=(n_blocks,),
        in_specs=[
            pl.BlockSpec((B, D), lambda i: (i, 0)),
            pl.BlockSpec((B, 1), lambda i: (i, 0)),
            pl.BlockSpec((K, D), lambda i: (0, 0)),
        ],
        out_specs=[
            pl.BlockSpec((B, D2), lambda i: (i, 0)),
            pl.BlockSpec((1, 1), lambda i: (0, 0)),
        ],
        out_shape=[
            jax.ShapeDtypeStruct((_NPAD, D2), jnp.float32),
            jax.ShapeDtypeStruct((1, 1), jnp.float32),
        ],
    )(node_feat, score, codebook)

    idx_pad = jnp.pad(batch.astype(jnp.int32), (0, _NPAD - N),
                      constant_values=_PAD)
    idx_pad = idx_pad.reshape(_NT, _NCHUNK, _CHUNK)
    zer = jnp.zeros((64, D2), jnp.float32)

    mesh = plsc.VectorSubcoreMesh(core_axis_name="c", subcore_axis_name="s")
    pool = pl.kernel(
        _pool_sc_body,
        mesh=mesh,
        out_type=jax.ShapeDtypeStruct((_NC, G, D2), jnp.float32),
        scratch_types=[
            pltpu.VMEM((_CHUNK, D2), jnp.float32),
            pltpu.VMEM((_NCHUNK, _CHUNK), jnp.int32),
            pltpu.VMEM_SHARED((G + 1, D2), jnp.float32),
        ],
    )
    acc_p = pool(cr, idx_pad, zer)

    one_b = jnp.ones((_CHUNK, D2), jnp.float32)
    cntk = pl.kernel(
        _cnt_sc_body,
        mesh=plsc.VectorSubcoreMesh(core_axis_name="c", subcore_axis_name="s"),
        out_type=jax.ShapeDtypeStruct((_NC, G, D2), jnp.float32),
        scratch_types=[
            pltpu.VMEM((_CHUNK, D2), jnp.float32),
            pltpu.VMEM((_NCHUNK, _CHUNK), jnp.int32),
            pltpu.VMEM_SHARED((G + 1, D2), jnp.float32),
        ],
    )
    cnt_p = cntk(idx_pad, one_b, zer)

    logit, cg, sg, cmt_out = pl.pallas_call(
        functools.partial(_classifier_body, n_total=N, d=D),
        in_specs=[
            pl.BlockSpec((_NC, G, D2), lambda: (0, 0, 0)),
            pl.BlockSpec((_NC, G, D2), lambda: (0, 0, 0)),
            pl.BlockSpec((1, 1), lambda: (0, 0)),
            pl.BlockSpec((D, H), lambda: (0, 0)),
            pl.BlockSpec((1, H), lambda: (0, 0)),
            pl.BlockSpec((1, H), lambda: (0, 0)),
            pl.BlockSpec((1, H), lambda: (0, 0)),
            pl.BlockSpec((H, 1), lambda: (0, 0)),
            pl.BlockSpec((1, 1), lambda: (0, 0)),
        ],
        out_specs=[
            pl.BlockSpec((G, 1), lambda: (0, 0)),
            pl.BlockSpec((G, D), lambda: (0, 0)),
            pl.BlockSpec((G, D), lambda: (0, 0)),
            pl.BlockSpec((1, 1), lambda: (0, 0)),
        ],
        out_shape=[
            jax.ShapeDtypeStruct((G, 1), jnp.float32),
            jax.ShapeDtypeStruct((G, D), jnp.float32),
            jax.ShapeDtypeStruct((G, D), jnp.float32),
            jax.ShapeDtypeStruct((1, 1), jnp.float32),
        ],
    )(acc_p, cnt_p, cmt, W1, b1.reshape(1, H), gamma.reshape(1, H),
      beta.reshape(1, H), W2, b2.reshape(1, 1))

    return (logit, cg, sg, cmt_out.reshape(()))


# speedup vs baseline: 1.0069x; 1.0069x over previous
import functools

import jax
import jax.numpy as jnp
from jax import lax
from jax.experimental import pallas as pl
from jax.experimental.pallas import tpu as pltpu
from jax.experimental.pallas import tpu_sc as plsc

G = 1024
COMMIT = 0.25

_NC = 2
_NS = 16
_NT = _NC * _NS
_CHUNK = 128
_NCHUNK = 25
_NPAD = _NT * _NCHUNK * _CHUNK
_PAD = G


def _vq_body(x_ref, sc_ref, cb_ref, cr_ref, cmt_ref):
    i = pl.program_id(0)

    @pl.when(i == 0)
    def _init():
        cmt_ref[...] = jnp.zeros_like(cmt_ref)

    x = x_ref[...]
    cb = cb_ref[...]
    K = cb.shape[0]

    x_sq = jnp.sum(x * x, axis=1, keepdims=True)
    cb_sq = jnp.sum(cb * cb, axis=1)[None, :]
    xc = lax.dot_general(x, cb, (((1,), (1,)), ((), ())),
                         preferred_element_type=jnp.float32)
    dist = x_sq - 2.0 * xc + cb_sq

    m = jnp.min(dist, axis=1, keepdims=True)
    cmt_ref[...] += jnp.sum(m).reshape(1, 1)

    onehot_k = (dist == m).astype(jnp.float32)
    nsel = jnp.sum(onehot_k, axis=1, keepdims=True)
    q = lax.dot_general(onehot_k, cb, (((1,), (0,)), ((), ())),
                        preferred_element_type=jnp.float32) / nsel

    score = sc_ref[...]
    r = x + q
    c = r * score
    cr_ref[...] = jnp.concatenate([c, r], axis=1)


def _cnt_sc_body(idx_hbm, one_hbm, zer_hbm, cnt_hbm, onesb, idxb, cnt_sh):
    cid = lax.axis_index("c")
    sid = lax.axis_index("s")
    wid = cid * _NS + sid

    r0 = sid * 64
    pltpu.sync_copy(zer_hbm, cnt_sh.at[pl.ds(r0, 64)])

    @pl.when(sid == _NS - 1)
    def _pad_row():
        pltpu.sync_copy(zer_hbm.at[pl.ds(0, 1)], cnt_sh.at[pl.ds(G, 1)])

    pltpu.sync_copy(idx_hbm.at[wid], idxb)
    pltpu.sync_copy(one_hbm, onesb)

    plsc.subcore_barrier()

    for t in range(_NCHUNK):
        pltpu.sync_copy(onesb, cnt_sh.at[idxb.at[t]], add=True)

    plsc.subcore_barrier()

    pltpu.sync_copy(cnt_sh.at[pl.ds(r0, 64)], cnt_hbm.at[cid, pl.ds(r0, 64)])


def _pool_sc_body(cr_hbm, idx_hbm, zer_hbm, acc_hbm, buf, idxb, acc_sh):
    cid = lax.axis_index("c")
    sid = lax.axis_index("s")
    wid = cid * _NS + sid

    r0 = sid * 64
    pltpu.sync_copy(zer_hbm, acc_sh.at[pl.ds(r0, 64)])

    @pl.when(sid == _NS - 1)
    def _pad_row():
        pltpu.sync_copy(zer_hbm.at[pl.ds(0, 1)], acc_sh.at[pl.ds(G, 1)])

    pltpu.sync_copy(idx_hbm.at[wid], idxb)

    plsc.subcore_barrier()

    base = wid * (_NCHUNK * _CHUNK)
    for t in range(_NCHUNK):
        row0 = base + t * _CHUNK
        pltpu.sync_copy(cr_hbm.at[pl.ds(row0, _CHUNK)], buf)
        pltpu.sync_copy(buf, acc_sh.at[idxb.at[t]], add=True)

    plsc.subcore_barrier()

    pltpu.sync_copy(acc_sh.at[pl.ds(r0, 64)], acc_hbm.at[cid, pl.ds(r0, 64)])


def _classifier_body(acc_ref, cnt_ref, cmt_ref, w1_ref, b1_ref, g_ref, be_ref,
                     w2_ref, b2_ref, logit_ref, cg_ref, sg_ref, cmt_out_ref,
                     *, n_total, d):
    acc = acc_ref[0] + acc_ref[1]
    denom = jnp.maximum(cnt_ref[0, :, :1] + cnt_ref[1, :, :1], 1.0)
    cg = acc[:, :d] / denom
    rg = acc[:, d:] / denom
    sg = rg - cg
    cg_ref[...] = cg
    sg_ref[...] = sg

    h = lax.dot_general(cg, w1_ref[...], (((1,), (0,)), ((), ())),
                        preferred_element_type=jnp.float32) + b1_ref[...]
    gn = h.shape[0]
    mu = jnp.sum(h, axis=0, keepdims=True) / gn
    var = jnp.sum((h - mu) ** 2, axis=0, keepdims=True) / gn
    hn = (h - mu) / jnp.sqrt(var + 1e-5) * g_ref[...] + be_ref[...]
    hr = jnp.maximum(hn, 0.0)
    logit_ref[...] = lax.dot_general(hr, w2_ref[...], (((1,), (0,)), ((), ())),
                                     preferred_element_type=jnp.float32) + b2_ref[...]

    cmt_out_ref[...] = cmt_ref[...] * (COMMIT / (n_total * d))


def kernel(node_feat, score, codebook, W1, b1, gamma, beta, W2, b2, batch):
    N, D = node_feat.shape
    K = codebook.shape[0]
    H = W1.shape[1]
    D2 = 2 * D

    B = 4000
    n_blocks = N // B

    cr, cmt = pl.pallas_call(
        _vq_body,
        grid=(n_blocks,),
        in_specs=[
            pl.BlockSpec((B, D), lambda i: (i, 0)),
            pl.BlockSpec((B, 1), lambda i: (i, 0)),
            pl.BlockSpec((K, D), lambda i: (0, 0)),
        ],
        out_specs=[
            pl.BlockSpec((B, D2), lambda i: (i, 0)),
            pl.BlockSpec((1, 1), lambda i: (0, 0)),
        ],
        out_shape=[
            jax.ShapeDtypeStruct((_NPAD, D2), jnp.float32),
            jax.ShapeDtypeStruct((1, 1), jnp.float32),
        ],
    )(node_feat, score, codebook)

    idx_pad = jnp.pad(batch.astype(jnp.int32), (0, _NPAD - N),
                      constant_values=_PAD)
    idx_pad = idx_pad.reshape(_NT, _NCHUNK, _CHUNK)
    zer = jnp.zeros((64, D2), jnp.float32)

    mesh = plsc.VectorSubcoreMesh(core_axis_name="c", subcore_axis_name="s")
    pool = pl.kernel(
        _pool_sc_body,
        mesh=mesh,
        out_type=jax.ShapeDtypeStruct((_NC, G, D2), jnp.float32),
        scratch_types=[
            pltpu.VMEM((_CHUNK, D2), jnp.float32),
            pltpu.VMEM((_NCHUNK, _CHUNK), jnp.int32),
            pltpu.VMEM_SHARED((G + 1, D2), jnp.float32),
        ],
    )
    acc_p = pool(cr, idx_pad, zer)

    one_b = jnp.ones((_CHUNK, D2), jnp.float32)
    cntk = pl.kernel(
        _cnt_sc_body,
        mesh=plsc.VectorSubcoreMesh(core_axis_name="c", subcore_axis_name="s"),
        out_type=jax.ShapeDtypeStruct((_NC, G, D2), jnp.float32),
        scratch_types=[
            pltpu.VMEM((_CHUNK, D2), jnp.float32),
            pltpu.VMEM((_NCHUNK, _CHUNK), jnp.int32),
            pltpu.VMEM_SHARED((G + 1, D2), jnp.float32),
        ],
    )
    cnt_p = cntk(idx_pad, one_b, zer)

    logit, cg, sg, cmt_out = pl.pallas_call(
        functools.partial(_classifier_body, n_total=N, d=D),
        in_specs=[
            pl.BlockSpec((_NC, G, D2), lambda: (0, 0, 0)),
            pl.BlockSpec((_NC, G, D2), lambda: (0, 0, 0)),
            pl.BlockSpec((1, 1), lambda: (0, 0)),
            pl.BlockSpec((D, H), lambda: (0, 0)),
            pl.BlockSpec((1, H), lambda: (0, 0)),
            pl.BlockSpec((1, H), lambda: (0, 0)),
            pl.BlockSpec((1, H), lambda: (0, 0)),
            pl.BlockSpec((H, 1), lambda: (0, 0)),
            pl.BlockSpec((1, 1), lambda: (0, 0)),
        ],
        out_specs=[
            pl.BlockSpec((G, 1), lambda: (0, 0)),
            pl.BlockSpec((G, D), lambda: (0, 0)),
            pl.BlockSpec((G, D), lambda: (0, 0)),
            pl.BlockSpec((1, 1), lambda: (0, 0)),
        ],
        out_shape=[
            jax.ShapeDtypeStruct((G, 1), jnp.float32),
            jax.ShapeDtypeStruct((G, D), jnp.float32),
            jax.ShapeDtypeStruct((G, D), jnp.float32),
            jax.ShapeDtypeStruct((1, 1), jnp.float32),
        ],
    )(acc_p, cnt_p, cmt, W1, b1.reshape(1, H), gamma.reshape(1, H),
      beta.reshape(1, H), W2, b2.reshape(1, 1))

    return (logit, cg, sg, cmt_out.reshape(()))


# speedup vs baseline: 1.0485x; 1.0413x over previous
import functools

import jax
import jax.numpy as jnp
from jax import lax
from jax.experimental import pallas as pl
from jax.experimental.pallas import tpu as pltpu
from jax.experimental.pallas import tpu_sc as plsc

G = 1024
COMMIT = 0.25

_NC = 2
_NS = 16
_NT = _NC * _NS
_CHUNK = 128
_NCHUNK = 25
_NPAD = _NT * _NCHUNK * _CHUNK
_PAD = G


def _vq_body(x_ref, sc_ref, cb_ref, cr_ref, cmt_ref):
    i = pl.program_id(0)

    @pl.when(i == 0)
    def _init():
        cmt_ref[...] = jnp.zeros_like(cmt_ref)

    x = x_ref[...]
    cb = cb_ref[...]
    K = cb.shape[0]

    x_sq = jnp.sum(x * x, axis=1, keepdims=True)
    cb_sq = jnp.sum(cb * cb, axis=1)[None, :]
    xc = lax.dot_general(x, cb, (((1,), (1,)), ((), ())),
                         preferred_element_type=jnp.float32)
    dist = x_sq - 2.0 * xc + cb_sq

    m = jnp.min(dist, axis=1, keepdims=True)
    cmt_ref[...] += jnp.sum(m).reshape(1, 1)

    onehot_k = (dist == m).astype(jnp.float32)
    q = lax.dot_general(onehot_k, cb, (((1,), (0,)), ((), ())),
                        preferred_element_type=jnp.float32)

    score = sc_ref[...]
    r = x + q
    c = r * score
    cr_ref[...] = jnp.concatenate([c, r], axis=1)


def _cnt_sc_body(idx_hbm, one_hbm, zer_hbm, cnt_hbm, onesb, idxb, cnt_sh):
    cid = lax.axis_index("c")
    sid = lax.axis_index("s")
    wid = cid * _NS + sid

    r0 = sid * 64
    pltpu.sync_copy(zer_hbm, cnt_sh.at[pl.ds(r0, 64)])

    @pl.when(sid == _NS - 1)
    def _pad_row():
        pltpu.sync_copy(zer_hbm.at[pl.ds(0, 1)], cnt_sh.at[pl.ds(G, 1)])

    pltpu.sync_copy(idx_hbm.at[wid], idxb)
    pltpu.sync_copy(one_hbm, onesb)

    plsc.subcore_barrier()

    for t in range(_NCHUNK):
        pltpu.sync_copy(onesb, cnt_sh.at[idxb.at[t]], add=True)

    plsc.subcore_barrier()

    pltpu.sync_copy(cnt_sh.at[pl.ds(r0, 64)], cnt_hbm.at[cid, pl.ds(r0, 64)])


def _pool_sc_body(cr_hbm, idx_hbm, zer_hbm, acc_hbm, buf, idxb, acc_sh):
    cid = lax.axis_index("c")
    sid = lax.axis_index("s")
    wid = cid * _NS + sid

    r0 = sid * 64
    pltpu.sync_copy(zer_hbm, acc_sh.at[pl.ds(r0, 64)])

    @pl.when(sid == _NS - 1)
    def _pad_row():
        pltpu.sync_copy(zer_hbm.at[pl.ds(0, 1)], acc_sh.at[pl.ds(G, 1)])

    pltpu.sync_copy(idx_hbm.at[wid], idxb)

    plsc.subcore_barrier()

    base = wid * (_NCHUNK * _CHUNK)
    for t in range(_NCHUNK):
        row0 = base + t * _CHUNK
        pltpu.sync_copy(cr_hbm.at[pl.ds(row0, _CHUNK)], buf)
        pltpu.sync_copy(buf, acc_sh.at[idxb.at[t]], add=True)

    plsc.subcore_barrier()

    pltpu.sync_copy(acc_sh.at[pl.ds(r0, 64)], acc_hbm.at[cid, pl.ds(r0, 64)])


def _classifier_body(acc_ref, cnt_ref, cmt_ref, w1_ref, b1_ref, g_ref, be_ref,
                     w2_ref, b2_ref, logit_ref, cg_ref, sg_ref, cmt_out_ref,
                     *, n_total, d):
    acc = acc_ref[0] + acc_ref[1]
    denom = jnp.maximum(cnt_ref[0, :, :1] + cnt_ref[1, :, :1], 1.0)
    cg = acc[:, :d] / denom
    rg = acc[:, d:] / denom
    sg = rg - cg
    cg_ref[...] = cg
    sg_ref[...] = sg

    h = lax.dot_general(cg, w1_ref[...], (((1,), (0,)), ((), ())),
                        preferred_element_type=jnp.float32) + b1_ref[...]
    gn = h.shape[0]
    mu = jnp.sum(h, axis=0, keepdims=True) / gn
    var = jnp.sum((h - mu) ** 2, axis=0, keepdims=True) / gn
    hn = (h - mu) / jnp.sqrt(var + 1e-5) * g_ref[...] + be_ref[...]
    hr = jnp.maximum(hn, 0.0)
    logit_ref[...] = lax.dot_general(hr, w2_ref[...], (((1,), (0,)), ((), ())),
                                     preferred_element_type=jnp.float32) + b2_ref[...]

    cmt_out_ref[...] = cmt_ref[...] * (COMMIT / (n_total * d))


def kernel(node_feat, score, codebook, W1, b1, gamma, beta, W2, b2, batch):
    N, D = node_feat.shape
    K = codebook.shape[0]
    H = W1.shape[1]
    D2 = 2 * D

    B = 4000
    n_blocks = N // B

    cr, cmt = pl.pallas_call(
        _vq_body,
        grid=(n_blocks,),
        in_specs=[
            pl.BlockSpec((B, D), lambda i: (i, 0)),
            pl.BlockSpec((B, 1), lambda i: (i, 0)),
            pl.BlockSpec((K, D), lambda i: (0, 0)),
        ],
        out_specs=[
            pl.BlockSpec((B, D2), lambda i: (i, 0)),
            pl.BlockSpec((1, 1), lambda i: (0, 0)),
        ],
        out_shape=[
            jax.ShapeDtypeStruct((_NPAD, D2), jnp.float32),
            jax.ShapeDtypeStruct((1, 1), jnp.float32),
        ],
    )(node_feat, score, codebook)

    idx_pad = jnp.pad(batch.astype(jnp.int32), (0, _NPAD - N),
                      constant_values=_PAD)
    idx_pad = idx_pad.reshape(_NT, _NCHUNK, _CHUNK)
    zer = jnp.zeros((64, D2), jnp.float32)

    mesh = plsc.VectorSubcoreMesh(core_axis_name="c", subcore_axis_name="s")
    pool = pl.kernel(
        _pool_sc_body,
        mesh=mesh,
        out_type=jax.ShapeDtypeStruct((_NC, G, D2), jnp.float32),
        scratch_types=[
            pltpu.VMEM((_CHUNK, D2), jnp.float32),
            pltpu.VMEM((_NCHUNK, _CHUNK), jnp.int32),
            pltpu.VMEM_SHARED((G + 1, D2), jnp.float32),
        ],
    )
    acc_p = pool(cr, idx_pad, zer)

    one_b = jnp.ones((_CHUNK, D2), jnp.float32)
    cntk = pl.kernel(
        _cnt_sc_body,
        mesh=plsc.VectorSubcoreMesh(core_axis_name="c", subcore_axis_name="s"),
        out_type=jax.ShapeDtypeStruct((_NC, G, D2), jnp.float32),
        scratch_types=[
            pltpu.VMEM((_CHUNK, D2), jnp.float32),
            pltpu.VMEM((_NCHUNK, _CHUNK), jnp.int32),
            pltpu.VMEM_SHARED((G + 1, D2), jnp.float32),
        ],
    )
    cnt_p = cntk(idx_pad, one_b, zer)

    logit, cg, sg, cmt_out = pl.pallas_call(
        functools.partial(_classifier_body, n_total=N, d=D),
        in_specs=[
            pl.BlockSpec((_NC, G, D2), lambda: (0, 0, 0)),
            pl.BlockSpec((_NC, G, D2), lambda: (0, 0, 0)),
            pl.BlockSpec((1, 1), lambda: (0, 0)),
            pl.BlockSpec((D, H), lambda: (0, 0)),
            pl.BlockSpec((1, H), lambda: (0, 0)),
            pl.BlockSpec((1, H), lambda: (0, 0)),
            pl.BlockSpec((1, H), lambda: (0, 0)),
            pl.BlockSpec((H, 1), lambda: (0, 0)),
            pl.BlockSpec((1, 1), lambda: (0, 0)),
        ],
        out_specs=[
            pl.BlockSpec((G, 1), lambda: (0, 0)),
            pl.BlockSpec((G, D), lambda: (0, 0)),
            pl.BlockSpec((G, D), lambda: (0, 0)),
            pl.BlockSpec((1, 1), lambda: (0, 0)),
        ],
        out_shape=[
            jax.ShapeDtypeStruct((G, 1), jnp.float32),
            jax.ShapeDtypeStruct((G, D), jnp.float32),
            jax.ShapeDtypeStruct((G, D), jnp.float32),
            jax.ShapeDtypeStruct((1, 1), jnp.float32),
        ],
    )(acc_p, cnt_p, cmt, W1, b1.reshape(1, H), gamma.reshape(1, H),
      beta.reshape(1, H), W2, b2.reshape(1, 1))

    return (logit, cg, sg, cmt_out.reshape(()))


# speedup vs baseline: 1.0600x; 1.0110x over previous
import functools

import jax
import jax.numpy as jnp
from jax import lax
from jax.experimental import pallas as pl
from jax.experimental.pallas import tpu as pltpu
from jax.experimental.pallas import tpu_sc as plsc

G = 1024
COMMIT = 0.25

_NC = 2
_NS = 16
_NT = _NC * _NS
_CHUNK = 128
_NCHUNK = 25
_NPAD = _NT * _NCHUNK * _CHUNK
_PAD = G


def _vq_body(x_ref, sc_ref, cb_ref, cr_ref, cmt_ref):
    i = pl.program_id(0)

    @pl.when(i == 0)
    def _init():
        cmt_ref[...] = jnp.zeros_like(cmt_ref)

    x = x_ref[...]
    cb = cb_ref[...]
    K = cb.shape[0]

    x_sq = jnp.sum(x * x, axis=1, keepdims=True)
    cb_sq = jnp.sum(cb * cb, axis=1)[None, :]
    xc = lax.dot_general(x, cb, (((1,), (1,)), ((), ())),
                         preferred_element_type=jnp.float32)
    dist = x_sq - 2.0 * xc + cb_sq

    m = jnp.min(dist, axis=1, keepdims=True)
    cmt_ref[...] += jnp.sum(m).reshape(1, 1)

    onehot_k = (dist == m).astype(jnp.float32)
    q = lax.dot_general(onehot_k, cb, (((1,), (0,)), ((), ())),
                        preferred_element_type=jnp.float32)

    score = sc_ref[...]
    r = x + q
    c = r * score
    cr_ref[...] = jnp.concatenate([c, r], axis=1)


def _cnt_sc_body(idx_hbm, one_hbm, zer_hbm, cnt_hbm, onesb, idxb, cnt_sh):
    cid = lax.axis_index("c")
    sid = lax.axis_index("s")
    wid = cid * _NS + sid

    r0 = sid * 64
    pltpu.sync_copy(zer_hbm, cnt_sh.at[pl.ds(r0, 64)])

    @pl.when(sid == _NS - 1)
    def _pad_row():
        pltpu.sync_copy(zer_hbm.at[pl.ds(0, 1)], cnt_sh.at[pl.ds(G, 1)])

    pltpu.sync_copy(idx_hbm.at[wid], idxb)
    pltpu.sync_copy(one_hbm, onesb)

    plsc.subcore_barrier()

    for t in range(_NCHUNK):
        pltpu.sync_copy(onesb, cnt_sh.at[idxb.at[t]], add=True)

    plsc.subcore_barrier()

    pltpu.sync_copy(cnt_sh.at[pl.ds(r0, 64)], cnt_hbm.at[cid, pl.ds(r0, 64)])


def _pool_sc_body(cr_hbm, idx_hbm, zer_hbm, acc_hbm, buf, idxb, acc_sh):
    cid = lax.axis_index("c")
    sid = lax.axis_index("s")
    wid = cid * _NS + sid

    r0 = sid * 64
    pltpu.sync_copy(zer_hbm, acc_sh.at[pl.ds(r0, 64)])

    @pl.when(sid == _NS - 1)
    def _pad_row():
        pltpu.sync_copy(zer_hbm.at[pl.ds(0, 1)], acc_sh.at[pl.ds(G, 1)])

    pltpu.sync_copy(idx_hbm.at[wid], idxb)

    plsc.subcore_barrier()

    base = wid * (_NCHUNK * _CHUNK)
    for t in range(_NCHUNK):
        row0 = base + t * _CHUNK
        pltpu.sync_copy(cr_hbm.at[pl.ds(row0, _CHUNK)], buf)
        pltpu.sync_copy(buf, acc_sh.at[idxb.at[t]], add=True)

    plsc.subcore_barrier()

    pltpu.sync_copy(acc_sh.at[pl.ds(r0, 64)], acc_hbm.at[cid, pl.ds(r0, 64)])


def _classifier_body(acc_ref, cnt_ref, cmt_ref, w1_ref, b1_ref, g_ref, be_ref,
                     w2_ref, b2_ref, logit_ref, cg_ref, sg_ref, cmt_out_ref,
                     *, n_total, d):
    acc = acc_ref[0] + acc_ref[1]
    denom = jnp.maximum(cnt_ref[0, :, :1] + cnt_ref[1, :, :1], 1.0)
    cg = acc[:, :d] / denom
    rg = acc[:, d:] / denom
    sg = rg - cg
    cg_ref[...] = cg
    sg_ref[...] = sg

    h = lax.dot_general(cg, w1_ref[...], (((1,), (0,)), ((), ())),
                        preferred_element_type=jnp.float32) + b1_ref[...]
    gn = h.shape[0]
    mu = jnp.sum(h, axis=0, keepdims=True) / gn
    var = jnp.sum((h - mu) ** 2, axis=0, keepdims=True) / gn
    hn = (h - mu) / jnp.sqrt(var + 1e-5) * g_ref[...] + be_ref[...]
    hr = jnp.maximum(hn, 0.0)
    logit_ref[...] = lax.dot_general(hr, w2_ref[...], (((1,), (0,)), ((), ())),
                                     preferred_element_type=jnp.float32) + b2_ref[...]

    cmt_out_ref[...] = cmt_ref[...] * (COMMIT / (n_total * d))


def kernel(node_feat, score, codebook, W1, b1, gamma, beta, W2, b2, batch):
    N, D = node_feat.shape
    K = codebook.shape[0]
    H = W1.shape[1]
    D2 = 2 * D

    B = 5000
    n_blocks = N // B

    cr, cmt = pl.pallas_call(
        _vq_body,
        grid=(n_blocks,),
        in_specs=[
            pl.BlockSpec((B, D), lambda i: (i, 0)),
            pl.BlockSpec((B, 1), lambda i: (i, 0)),
            pl.BlockSpec((K, D), lambda i: (0, 0)),
        ],
        out_specs=[
            pl.BlockSpec((B, D2), lambda i: (i, 0)),
            pl.BlockSpec((1, 1), lambda i: (0, 0)),
        ],
        out_shape=[
            jax.ShapeDtypeStruct((_NPAD, D2), jnp.float32),
            jax.ShapeDtypeStruct((1, 1), jnp.float32),
        ],
    )(node_feat, score, codebook)

    idx_pad = jnp.pad(batch.astype(jnp.int32), (0, _NPAD - N),
                      constant_values=_PAD)
    idx_pad = idx_pad.reshape(_NT, _NCHUNK, _CHUNK)
    zer = jnp.zeros((64, D2), jnp.float32)

    mesh = plsc.VectorSubcoreMesh(core_axis_name="c", subcore_axis_name="s")
    pool = pl.kernel(
        _pool_sc_body,
        mesh=mesh,
        out_type=jax.ShapeDtypeStruct((_NC, G, D2), jnp.float32),
        scratch_types=[
            pltpu.VMEM((_CHUNK, D2), jnp.float32),
            pltpu.VMEM((_NCHUNK, _CHUNK), jnp.int32),
            pltpu.VMEM_SHARED((G + 1, D2), jnp.float32),
        ],
    )
    acc_p = pool(cr, idx_pad, zer)

    one_b = jnp.ones((_CHUNK, D2), jnp.float32)
    cntk = pl.kernel(
        _cnt_sc_body,
        mesh=plsc.VectorSubcoreMesh(core_axis_name="c", subcore_axis_name="s"),
        out_type=jax.ShapeDtypeStruct((_NC, G, D2), jnp.float32),
        scratch_types=[
            pltpu.VMEM((_CHUNK, D2), jnp.float32),
            pltpu.VMEM((_NCHUNK, _CHUNK), jnp.int32),
            pltpu.VMEM_SHARED((G + 1, D2), jnp.float32),
        ],
    )
    cnt_p = cntk(idx_pad, one_b, zer)

    logit, cg, sg, cmt_out = pl.pallas_call(
        functools.partial(_classifier_body, n_total=N, d=D),
        in_specs=[
            pl.BlockSpec((_NC, G, D2), lambda: (0, 0, 0)),
            pl.BlockSpec((_NC, G, D2), lambda: (0, 0, 0)),
            pl.BlockSpec((1, 1), lambda: (0, 0)),
            pl.BlockSpec((D, H), lambda: (0, 0)),
            pl.BlockSpec((1, H), lambda: (0, 0)),
            pl.BlockSpec((1, H), lambda: (0, 0)),
            pl.BlockSpec((1, H), lambda: (0, 0)),
            pl.BlockSpec((H, 1), lambda: (0, 0)),
            pl.BlockSpec((1, 1), lambda: (0, 0)),
        ],
        out_specs=[
            pl.BlockSpec((G, 1), lambda: (0, 0)),
            pl.BlockSpec((G, D), lambda: (0, 0)),
            pl.BlockSpec((G, D), lambda: (0, 0)),
            pl.BlockSpec((1, 1), lambda: (0, 0)),
        ],
        out_shape=[
            jax.ShapeDtypeStruct((G, 1), jnp.float32),
            jax.ShapeDtypeStruct((G, D), jnp.float32),
            jax.ShapeDtypeStruct((G, D), jnp.float32),
            jax.ShapeDtypeStruct((1, 1), jnp.float32),
        ],
    )(acc_p, cnt_p, cmt, W1, b1.reshape(1, H), gamma.reshape(1, H),
      beta.reshape(1, H), W2, b2.reshape(1, 1))

    return (logit, cg, sg, cmt_out.reshape(()))
